# baseline (speedup 1.0000x reference)
import functools

import jax
import jax.numpy as jnp
from jax import lax
from jax.experimental import pallas as pl
from jax.experimental.pallas import tpu as pltpu
from jax.experimental.pallas import tpu_sc as plsc

BATCH = 16384
EMB = 16
CHUNK = 128


@functools.cache
def _build():
  info = plsc.get_sparse_core_info()
  nw = info.num_cores * info.num_subcores
  bpw = BATCH // nw
  nch = bpw // CHUNK
  mesh = plsc.VectorSubcoreMesh(
      core_axis_name="c", subcore_axis_name="s",
      num_cores=info.num_cores, num_subcores=info.num_subcores)

  @functools.partial(
      pl.kernel,
      mesh=mesh,
      compiler_params=pltpu.CompilerParams(use_tc_tiling_on_sc=False),
      out_type=jax.ShapeDtypeStruct((BATCH, EMB), jnp.float32),
      scratch_types=[
          pltpu.VMEM((nch, CHUNK), jnp.int32),
          pltpu.VMEM((nch, CHUNK), jnp.int32),
          pltpu.VMEM((bpw, EMB), jnp.float32),
          pltpu.VMEM((bpw, EMB), jnp.float32),
          pltpu.SemaphoreType.DMA,
          pltpu.SemaphoreType.DMA,
      ],
  )
  def gmf(uidx_hbm, iidx_hbm, uemb_hbm, iemb_hbm, out_hbm,
          uidx_v, iidx_v, u_v, i_v, sem_idx, sem_rows):
    wid = lax.axis_index("s") * info.num_cores + lax.axis_index("c")
    base = wid * bpw

    idx_copies = []
    for j in range(nch):
      idx_copies.append(
          pltpu.async_copy(uidx_hbm.at[pl.ds(base + j * CHUNK, CHUNK)],
                           uidx_v.at[j], sem_idx))
      idx_copies.append(
          pltpu.async_copy(iidx_hbm.at[pl.ds(base + j * CHUNK, CHUNK)],
                           iidx_v.at[j], sem_idx))
    for c in idx_copies:
      c.wait()

    row_copies = []
    for j in range(nch):
      row_copies.append(
          pltpu.async_copy(uemb_hbm.at[uidx_v.at[j]],
                           u_v.at[pl.ds(j * CHUNK, CHUNK)], sem_rows))
      row_copies.append(
          pltpu.async_copy(iemb_hbm.at[iidx_v.at[j]],
                           i_v.at[pl.ds(j * CHUNK, CHUNK)], sem_rows))
    for c in row_copies:
      c.wait()

    def mul_body(r):
      u_v[r] = u_v[r] * i_v[r]

    plsc.parallel_loop(0, bpw, 1, unroll=8)(mul_body)

    pltpu.sync_copy(u_v, out_hbm.at[pl.ds(base, bpw)])

  return gmf


def kernel(user_idx, item_idx, user_emb, item_emb):
  return _build()(user_idx.astype(jnp.int32), item_idx.astype(jnp.int32),
                  user_emb, item_emb)


# speedup vs baseline: 5.1277x; 5.1277x over previous
import functools

import jax
import jax.numpy as jnp
from jax import lax
from jax.experimental import pallas as pl
from jax.experimental.pallas import tpu as pltpu
from jax.experimental.pallas import tpu_sc as plsc

BATCH = 16384
EMB = 16
NROWS = 1000000
GRP = 16
RING = 8
TAIL = NROWS - 128
CMAX = TAIL // 128


@functools.cache
def _build():
  info = plsc.get_sparse_core_info()
  nw = info.num_cores * info.num_subcores
  bpw = BATCH // nw
  ngrp = bpw // GRP
  mesh = plsc.VectorSubcoreMesh(
      core_axis_name="c", subcore_axis_name="s",
      num_cores=info.num_cores, num_subcores=info.num_subcores)

  @functools.partial(
      pl.kernel,
      mesh=mesh,
      compiler_params=pltpu.CompilerParams(
          use_tc_tiling_on_sc=True, needs_layout_passes=False),
      out_type=jax.ShapeDtypeStruct((EMB, BATCH), jnp.float32),
      scratch_types=[
          pltpu.VMEM((bpw,), jnp.int32),
          pltpu.VMEM((bpw,), jnp.int32),
          pltpu.VMEM((RING, EMB, 128), jnp.float32),
          pltpu.VMEM((RING, EMB, 128), jnp.float32),
          pltpu.VMEM((EMB, 128), jnp.float32),
          pltpu.VMEM((EMB, 128), jnp.float32),
          pltpu.VMEM((EMB, bpw), jnp.float32),
          *([pltpu.SemaphoreType.DMA] * (2 * RING + 1)),
      ],
  )
  def gmf(uidx_hbm, iidx_hbm, uemb_hbm, iemb_hbm, utail_hbm, itail_hbm,
          out_hbm, uidx_v, iidx_v, u_ring, i_ring, utail_v, itail_v, p_v,
          *sems):
    sem_u = sems[:RING]
    sem_i = sems[RING:2 * RING]
    sem_idx = sems[2 * RING]
    wid = lax.axis_index("s") * info.num_cores + lax.axis_index("c")
    base = wid * bpw

    cu = pltpu.async_copy(uidx_hbm.at[pl.ds(base, bpw)], uidx_v, sem_idx)
    ci = pltpu.async_copy(iidx_hbm.at[pl.ds(base, bpw)], iidx_v, sem_idx)
    pltpu.sync_copy(utail_hbm, utail_v)
    pltpu.sync_copy(itail_hbm, itail_v)
    cu.wait()
    ci.wait()

    lanes = lax.iota(jnp.int32, 16)

    def fire(ue, ie, s):
      uc = jnp.minimum(ue >> 7, CMAX) * 128
      ic = jnp.minimum(ie >> 7, CMAX) * 128
      pltpu.async_copy(
          uemb_hbm.at[:, pl.ds(pl.multiple_of(uc, 128), 128)],
          u_ring.at[s], sem_u[s])
      pltpu.async_copy(
          iemb_hbm.at[:, pl.ds(pl.multiple_of(ic, 128), 128)],
          i_ring.at[s], sem_i[s])

    def pick(ring, tail_v, idx):
      inring = plsc.load_gather(
          ring, [lanes, jnp.full((16,), idx & 127, jnp.int32)])
      lt = jnp.maximum(idx - TAIL, 0)
      intail = plsc.load_gather(
          tail_v, [lanes, jnp.full((16,), lt, jnp.int32)])
      return jnp.where(jnp.full((16,), idx >= TAIL, jnp.bool_),
                       intail, inring)

    def extract(pu, pi, j, s):
      pltpu.make_async_copy(uemb_hbm.at[:, pl.ds(0, 128)],
                            u_ring.at[s], sem_u[s]).wait()
      pltpu.make_async_copy(iemb_hbm.at[:, pl.ds(0, 128)],
                            i_ring.at[s], sem_i[s]).wait()
      uvec = pick(u_ring.at[s], utail_v, pu)
      ivec = pick(i_ring.at[s], itail_v, pi)
      plsc.store_scatter(p_v, [lanes, jnp.full((16,), j, jnp.int32)],
                         uvec * ivec)

    ug0 = uidx_v[pl.ds(0, GRP)]
    ig0 = iidx_v[pl.ds(0, GRP)]
    for e in range(GRP):
      if e >= RING:
        extract(ug0[e - RING], ig0[e - RING], e - RING, e % RING)
      fire(ug0[e], ig0[e], e % RING)

    def outer(g, carry):
      pu, pi = carry
      ug = uidx_v[pl.ds(g * GRP, GRP)]
      ig = iidx_v[pl.ds(g * GRP, GRP)]
      for e in range(GRP):
        j = g * GRP + e
        lx = (e + RING) % GRP
        vu, vi = (pu, pi) if e < RING else (ug, ig)
        extract(vu[lx], vi[lx], j - RING, e % RING)
        fire(ug[e], ig[e], e % RING)
      return ug, ig

    pu, pi = lax.fori_loop(1, ngrp, outer, (ug0, ig0))

    for e in range(RING):
      lx = e + RING
      extract(pu[lx], pi[lx], (ngrp - 1) * GRP + lx, e)

    pltpu.sync_copy(p_v, out_hbm.at[:, pl.ds(base, bpw)])

  return gmf


def kernel(user_idx, item_idx, user_emb, item_emb):
  u_t = user_emb.T
  i_t = item_emb.T
  out_t = _build()(user_idx.astype(jnp.int32), item_idx.astype(jnp.int32),
                   u_t, i_t, u_t[:, TAIL:], i_t[:, TAIL:])
  return out_t.T


# speedup vs baseline: 5.3408x; 1.0416x over previous
import functools

import jax
import jax.numpy as jnp
from jax import lax
from jax.experimental import pallas as pl
from jax.experimental.pallas import tpu as pltpu
from jax.experimental.pallas import tpu_sc as plsc

BATCH = 16384
EMB = 16
NROWS = 1000000
GRP = 16
RING = 8
TAIL = NROWS - 128
CMAX = TAIL // 128


@functools.cache
def _build():
  info = plsc.get_sparse_core_info()
  nw = info.num_cores * info.num_subcores
  bpw = BATCH // nw
  ngrp = bpw // GRP
  mesh = plsc.VectorSubcoreMesh(
      core_axis_name="c", subcore_axis_name="s",
      num_cores=info.num_cores, num_subcores=info.num_subcores)

  @functools.partial(
      pl.kernel,
      mesh=mesh,
      compiler_params=pltpu.CompilerParams(
          use_tc_tiling_on_sc=True, needs_layout_passes=False),
      out_type=jax.ShapeDtypeStruct((EMB, BATCH), jnp.float32),
      scratch_types=[
          pltpu.VMEM((bpw,), jnp.int32),
          pltpu.VMEM((bpw,), jnp.int32),
          pltpu.VMEM((RING, EMB, 128), jnp.float32),
          pltpu.VMEM((RING, EMB, 128), jnp.float32),
          pltpu.VMEM((EMB, 128), jnp.float32),
          pltpu.VMEM((EMB, 128), jnp.float32),
          pltpu.VMEM((EMB, bpw), jnp.float32),
          pltpu.VMEM_SHARED((16, RING, EMB, 128), jnp.float32),
          *([pltpu.SemaphoreType.DMA] * (3 * RING + 1)),
      ],
  )
  def gmf(uidx_hbm, iidx_hbm, uemb_hbm, iemb_hbm, utail_hbm, itail_hbm,
          out_hbm, uidx_v, iidx_v, u_ring, i_vring, utail_v, itail_v, p_v,
          i_sh, *sems):
    sem_u = sems[:RING]
    sem_i = sems[RING:2 * RING]
    sem_x = sems[2 * RING:3 * RING]
    sem_idx = sems[3 * RING]
    sid = lax.axis_index("s")
    wid = sid * info.num_cores + lax.axis_index("c")
    base = wid * bpw

    cu = pltpu.async_copy(uidx_hbm.at[pl.ds(base, bpw)], uidx_v, sem_idx)
    ci = pltpu.async_copy(iidx_hbm.at[pl.ds(base, bpw)], iidx_v, sem_idx)
    pltpu.sync_copy(utail_hbm, utail_v)
    pltpu.sync_copy(itail_hbm, itail_v)
    cu.wait()
    ci.wait()

    lanes = lax.iota(jnp.int32, 16)

    def fire(ue, ie, s):
      uc = jnp.minimum(ue >> 7, CMAX) * 128
      ic = jnp.minimum(ie >> 7, CMAX) * 128
      pltpu.async_copy(
          uemb_hbm.at[:, pl.ds(pl.multiple_of(uc, 128), 128)],
          u_ring.at[s], sem_u[s])
      pltpu.async_copy(
          iemb_hbm.at[:, pl.ds(pl.multiple_of(ic, 128), 128)],
          i_sh.at[sid, s], sem_i[s])

    def hop(s):
      pltpu.make_async_copy(uemb_hbm.at[:, pl.ds(0, 128)],
                            u_ring.at[s], sem_u[s]).wait()
      pltpu.make_async_copy(uemb_hbm.at[:, pl.ds(0, 128)],
                            i_sh.at[sid, s], sem_i[s]).wait()
      pltpu.async_copy(i_sh.at[sid, s], i_vring.at[s], sem_x[s])

    def pick(ring, tail_v, idx):
      inring = plsc.load_gather(
          ring, [lanes, jnp.full((16,), idx & 127, jnp.int32)])
      lt = jnp.maximum(idx - TAIL, 0)
      intail = plsc.load_gather(
          tail_v, [lanes, jnp.full((16,), lt, jnp.int32)])
      return jnp.where(jnp.full((16,), idx >= TAIL, jnp.bool_),
                       intail, inring)

    def extract(pu, pi, j, s):
      pltpu.make_async_copy(i_sh.at[sid, s], i_vring.at[s],
                            sem_x[s]).wait()
      uvec = pick(u_ring.at[s], utail_v, pu)
      ivec = pick(i_vring.at[s], itail_v, pi)
      plsc.store_scatter(p_v, [lanes, jnp.full((16,), j, jnp.int32)],
                         uvec * ivec)

    ug0 = uidx_v[pl.ds(0, GRP)]
    ig0 = iidx_v[pl.ds(0, GRP)]
    for e in range(GRP):
      if e >= 4:
        hop((e + 4) % RING)
      if e >= RING:
        extract(ug0[e - RING], ig0[e - RING], e - RING, e % RING)
      fire(ug0[e], ig0[e], e % RING)

    def outer(g, carry):
      pu, pi = carry
      ug = uidx_v[pl.ds(g * GRP, GRP)]
      ig = iidx_v[pl.ds(g * GRP, GRP)]
      for e in range(GRP):
        j = g * GRP + e
        lx = (e + RING) % GRP
        vu, vi = (pu, pi) if e < RING else (ug, ig)
        hop((e + 4) % RING)
        extract(vu[lx], vi[lx], j - RING, e % RING)
        fire(ug[e], ig[e], e % RING)
      return ug, ig

    pu, pi = lax.fori_loop(1, ngrp, outer, (ug0, ig0))

    L = (ngrp - 1) * GRP
    for k in range(4):
      hop((12 + k) % RING)
    for k in range(RING):
      extract(pu[RING + k], pi[RING + k], L + RING + k, k)

    pltpu.sync_copy(p_v, out_hbm.at[:, pl.ds(base, bpw)])

  return gmf


def kernel(user_idx, item_idx, user_emb, item_emb):
  u_t = user_emb.T
  i_t = item_emb.T
  out_t = _build()(user_idx.astype(jnp.int32), item_idx.astype(jnp.int32),
                   u_t, i_t, u_t[:, TAIL:], i_t[:, TAIL:])
  return out_t.T


# speedup vs baseline: 6.4572x; 1.2090x over previous
import functools

import jax
import jax.numpy as jnp
from jax import lax
from jax.experimental import pallas as pl
from jax.experimental.pallas import tpu as pltpu
from jax.experimental.pallas import tpu_sc as plsc

BATCH = 16384
EMB = 16
NROWS = 1000000
GRP = 16
RING = 8
TAIL = NROWS - 128
CMAX = TAIL // 128


@functools.cache
def _build():
  info = plsc.get_sparse_core_info()
  nw = info.num_cores * info.num_subcores
  bpw = BATCH // nw
  ngrp = bpw // GRP
  mesh = plsc.VectorSubcoreMesh(
      core_axis_name="c", subcore_axis_name="s",
      num_cores=info.num_cores, num_subcores=info.num_subcores)

  @functools.partial(
      pl.kernel,
      mesh=mesh,
      compiler_params=pltpu.CompilerParams(
          use_tc_tiling_on_sc=True, needs_layout_passes=False),
      out_type=jax.ShapeDtypeStruct((EMB, BATCH), jnp.float32),
      scratch_types=[
          pltpu.VMEM((bpw,), jnp.int32),
          pltpu.VMEM((bpw,), jnp.int32),
          pltpu.VMEM((RING, EMB, 128), jnp.float32),
          pltpu.VMEM((RING, EMB, 128), jnp.float32),
          pltpu.VMEM((EMB, 128), jnp.float32),
          pltpu.VMEM((EMB, 128), jnp.float32),
          pltpu.VMEM((EMB, bpw), jnp.float32),
          pltpu.VMEM_SHARED((16, RING, EMB, 128), jnp.float32),
          *([pltpu.SemaphoreType.DMA] * (3 * RING + 1)),
      ],
  )
  def gmf(uidx_hbm, iidx_hbm, uemb_hbm, iemb_hbm, utail_hbm, itail_hbm,
          out_hbm, uidx_v, iidx_v, u_ring, i_vring, utail_v, itail_v, p_v,
          i_sh, *sems):
    sem_u = sems[:RING]
    sem_i = sems[RING:2 * RING]
    sem_x = sems[2 * RING:3 * RING]
    sem_idx = sems[3 * RING]
    sid = lax.axis_index("s")
    wid = sid * info.num_cores + lax.axis_index("c")
    base = wid * bpw

    cu = pltpu.async_copy(uidx_hbm.at[pl.ds(base, bpw)], uidx_v, sem_idx)
    ci = pltpu.async_copy(iidx_hbm.at[pl.ds(base, bpw)], iidx_v, sem_idx)
    pltpu.sync_copy(utail_hbm, utail_v)
    pltpu.sync_copy(itail_hbm, itail_v)
    cu.wait()
    ci.wait()

    lanes = lax.iota(jnp.int32, 16)

    def fire(ue, ie, s):
      uc = jnp.minimum(ue >> 7, CMAX) * 128
      ic = jnp.minimum(ie >> 7, CMAX) * 128
      pltpu.async_copy(
          uemb_hbm.at[:, pl.ds(pl.multiple_of(uc, 128), 128)],
          u_ring.at[s], sem_u[s])
      pltpu.async_copy(
          iemb_hbm.at[:, pl.ds(pl.multiple_of(ic, 128), 128)],
          i_sh.at[sid, s], sem_i[s])

    def hop(s):
      pltpu.make_async_copy(uemb_hbm.at[:, pl.ds(0, 128)],
                            u_ring.at[s], sem_u[s]).wait()
      pltpu.make_async_copy(uemb_hbm.at[:, pl.ds(0, 128)],
                            i_sh.at[sid, s], sem_i[s]).wait()
      pltpu.async_copy(i_sh.at[sid, s], i_vring.at[s], sem_x[s])

    def pick(ring, tail_v, idx):
      inring = plsc.load_gather(
          ring, [lanes, jnp.full((16,), idx & 127, jnp.int32)])
      lt = jnp.maximum(idx - TAIL, 0)
      intail = plsc.load_gather(
          tail_v, [lanes, jnp.full((16,), lt, jnp.int32)])
      return jnp.where(jnp.full((16,), idx >= TAIL, jnp.bool_),
                       intail, inring)

    def extract(pu, pi, j, s):
      pltpu.make_async_copy(i_sh.at[sid, s], i_vring.at[s],
                            sem_x[s]).wait()
      uvec = pick(u_ring.at[s], utail_v, pu)
      ivec = pick(i_vring.at[s], itail_v, pi)
      plsc.store_scatter(p_v, [lanes, jnp.full((16,), j, jnp.int32)],
                         uvec * ivec)

    ug0 = uidx_v[pl.ds(0, GRP)]
    ig0 = iidx_v[pl.ds(0, GRP)]
    for e in range(GRP):
      if e >= 6:
        hop((e + 2) % RING)
      if e >= RING:
        extract(ug0[e - RING], ig0[e - RING], e - RING, e % RING)
      fire(ug0[e], ig0[e], e % RING)

    def outer(g, carry):
      pu, pi = carry
      ug = uidx_v[pl.ds(g * GRP, GRP)]
      ig = iidx_v[pl.ds(g * GRP, GRP)]
      for e in range(GRP):
        j = g * GRP + e
        lx = (e + RING) % GRP
        vu, vi = (pu, pi) if e < RING else (ug, ig)
        hop((e + 2) % RING)
        extract(vu[lx], vi[lx], j - RING, e % RING)
        fire(ug[e], ig[e], e % RING)
      return ug, ig

    pu, pi = lax.fori_loop(1, ngrp, outer, (ug0, ig0))

    L = (ngrp - 1) * GRP
    for k in range(6):
      hop((10 + k) % RING)
    for k in range(RING):
      extract(pu[RING + k], pi[RING + k], L + RING + k, k)

    pltpu.sync_copy(p_v, out_hbm.at[:, pl.ds(base, bpw)])

  return gmf


def kernel(user_idx, item_idx, user_emb, item_emb):
  u_t = user_emb.T
  i_t = item_emb.T
  out_t = _build()(user_idx.astype(jnp.int32), item_idx.astype(jnp.int32),
                   u_t, i_t, u_t[:, TAIL:], i_t[:, TAIL:])
  return out_t.T
